# Initial kernel scaffold; baseline (speedup 1.0000x reference)
import jax
import jax.numpy as jnp
from jax.experimental import pallas as pl


def kernel(data, padding_mask, W1, W2):
    raise NotImplementedError("write your pallas kernel here")


# speedup vs baseline: 1.2721x; 1.2721x over previous
import jax
import jax.numpy as jnp
from jax.experimental import pallas as pl
from jax.experimental.pallas import tpu as pltpu

_NEG = -1e20


def _pool_kernel(x_ref, m_ref, w1_ref, w2_ref, o_ref):
    x = x_ref[0]
    m = m_ref[0]
    a = w2_ref.shape[1]
    h = jnp.tanh(jnp.dot(x, w1_ref[...], preferred_element_type=jnp.float32))
    logits = jnp.dot(h, w2_ref[...], preferred_element_type=jnp.float32)
    logits = logits * m + (1.0 - m) * _NEG
    mx = jnp.max(logits, axis=0, keepdims=True)
    p = jnp.exp(logits - mx)
    l = jnp.sum(p, axis=0, keepdims=True)
    acc = jax.lax.dot_general(p, x, (((0,), (0,)), ((), ())),
                              preferred_element_type=jnp.float32)
    winv = 1.0 / (l * float(a))
    o_ref[0] = jnp.dot(winv, acc, preferred_element_type=jnp.float32)


def kernel(data, padding_mask, W1, W2):
    B, S, H = data.shape
    U, A = W2.shape[0], W2.shape[1]
    mask3 = padding_mask[:, :, None]
    out = pl.pallas_call(
        _pool_kernel,
        out_shape=jax.ShapeDtypeStruct((B, 1, H), jnp.float32),
        grid=(B,),
        in_specs=[
            pl.BlockSpec((1, S, H), lambda b: (b, 0, 0)),
            pl.BlockSpec((1, S, 1), lambda b: (b, 0, 0)),
            pl.BlockSpec((H, U), lambda b: (0, 0)),
            pl.BlockSpec((U, A), lambda b: (0, 0)),
        ],
        out_specs=pl.BlockSpec((1, 1, H), lambda b: (b, 0, 0)),
        compiler_params=pltpu.CompilerParams(
            dimension_semantics=("parallel",),
            vmem_limit_bytes=56 * 1024 * 1024,
        ),
        name="self_attn_pool",
    )(data, mask3, W1, W2)
    return out.reshape(B, H)


# speedup vs baseline: 1.3152x; 1.0339x over previous
import jax
import jax.numpy as jnp
from jax.experimental import pallas as pl
from jax.experimental.pallas import tpu as pltpu


def _pool_kernel(x_ref, m_ref, w1_ref, w2_ref, o_ref):
    a = w2_ref.shape[1]
    w2 = w2_ref[...]
    bound = jnp.sum(jnp.abs(w2), axis=0, keepdims=True)
    h = jnp.tanh(jnp.dot(x_ref[...], w1_ref[...],
                         preferred_element_type=jnp.float32))
    logits = jnp.dot(h, w2, preferred_element_type=jnp.float32) + (-bound)
    p = jnp.exp(logits) * m_ref[0]
    l = jnp.sum(p, axis=0, keepdims=True)
    acc = jax.lax.dot_general(p, x_ref[...], (((0,), (0,)), ((), ())),
                              preferred_element_type=jnp.float32)
    winv = 1.0 / (l * float(a))
    o_ref[0] = jnp.dot(winv, acc, preferred_element_type=jnp.float32)


def kernel(data, padding_mask, W1, W2):
    B, S, H = data.shape
    U, A = W2.shape[0], W2.shape[1]
    data2 = data.reshape(B * S, H)
    mask3 = padding_mask.reshape(B, S, 1)
    out = pl.pallas_call(
        _pool_kernel,
        out_shape=jax.ShapeDtypeStruct((B, 1, H), jnp.float32),
        grid=(B,),
        in_specs=[
            pl.BlockSpec((S, H), lambda b: (b, 0)),
            pl.BlockSpec((1, S, 1), lambda b: (b, 0, 0)),
            pl.BlockSpec((H, U), lambda b: (0, 0)),
            pl.BlockSpec((U, A), lambda b: (0, 0)),
        ],
        out_specs=pl.BlockSpec((1, 1, H), lambda b: (b, 0, 0)),
        compiler_params=pltpu.CompilerParams(
            dimension_semantics=("parallel",),
            vmem_limit_bytes=56 * 1024 * 1024,
        ),
        name="self_attn_pool",
    )(data2, mask3, W1, W2)
    return out.reshape(B, H)


# speedup vs baseline: 1.3639x; 1.0370x over previous
import jax
import jax.numpy as jnp
from jax.experimental import pallas as pl
from jax.experimental.pallas import tpu as pltpu


def _pool_kernel(x1_ref, x2_ref, m_ref, w1_ref, w2_ref, o_ref):
    a = w2_ref.shape[1]
    sh = x1_ref.shape[0]
    w1 = w1_ref[...]
    w2 = w2_ref[...]
    bound = jnp.sum(jnp.abs(w2), axis=0, keepdims=True)
    acc = None
    l = None
    for i, x_ref in enumerate((x1_ref, x2_ref)):
        x = x_ref[...]
        h = jnp.tanh(jnp.dot(x, w1, preferred_element_type=jnp.float32))
        logits = jnp.dot(h, w2, preferred_element_type=jnp.float32) + (-bound)
        p = jnp.exp(logits) * m_ref[0, i * sh:(i + 1) * sh]
        li = jnp.sum(p, axis=0, keepdims=True)
        ai = jax.lax.dot_general(p, x, (((0,), (0,)), ((), ())),
                                 preferred_element_type=jnp.float32)
        acc = ai if acc is None else acc + ai
        l = li if l is None else l + li
    winv = 1.0 / (l * float(a))
    o_ref[0] = jnp.dot(winv, acc, preferred_element_type=jnp.float32)


def kernel(data, padding_mask, W1, W2):
    B, S, H = data.shape
    U, A = W2.shape[0], W2.shape[1]
    data2 = data.reshape(B * S, H)
    mask3 = padding_mask.reshape(B, S, 1)
    sh = S // 2
    out = pl.pallas_call(
        _pool_kernel,
        out_shape=jax.ShapeDtypeStruct((B, 1, H), jnp.float32),
        grid=(B,),
        in_specs=[
            pl.BlockSpec((sh, H), lambda b: (2 * b, 0)),
            pl.BlockSpec((sh, H), lambda b: (2 * b + 1, 0)),
            pl.BlockSpec((1, S, 1), lambda b: (b, 0, 0)),
            pl.BlockSpec((H, U), lambda b: (0, 0)),
            pl.BlockSpec((U, A), lambda b: (0, 0)),
        ],
        out_specs=pl.BlockSpec((1, 1, H), lambda b: (b, 0, 0)),
        compiler_params=pltpu.CompilerParams(
            dimension_semantics=("parallel",),
            vmem_limit_bytes=56 * 1024 * 1024,
        ),
        name="self_attn_pool",
    )(data2, data2, mask3, W1, W2)
    return out.reshape(B, H)
